# Initial kernel scaffold; baseline (speedup 1.0000x reference)
import jax
import jax.numpy as jnp
from jax.experimental import pallas as pl


def kernel(edge_index, user_table, item_table):
    raise NotImplementedError("write your pallas kernel here")


# speedup vs baseline: 5.3713x; 5.3713x over previous
import jax
import jax.numpy as jnp
from jax.experimental import pallas as pl

N_USERS = 50000
N_ITEMS = 50000
N_LAYERS = 3


def _finish(acc_ref, out_ref):
    out_ref[...] = acc_ref[...] * (1.0 / (N_LAYERS + 1))


def kernel(edge_index, user_table, item_table):
    x0 = jnp.concatenate([user_table, item_table], axis=0)
    num_nodes = N_USERS + N_ITEMS
    row = edge_index[0].astype(jnp.int32)
    col = edge_index[1].astype(jnp.int32) + N_USERS
    src = jnp.concatenate([row, col], axis=0)
    dst = jnp.concatenate([col, row], axis=0)
    deg = jnp.zeros((num_nodes,), jnp.float32).at[src].add(1.0)
    deg = jnp.clip(deg, 1.0, None)
    dis = deg ** -0.5
    norm_w = dis[src] * dis[dst]
    current = x0
    acc = x0
    for _ in range(N_LAYERS):
        msg = norm_w[:, None] * current[src]
        current = jnp.zeros((num_nodes, x0.shape[1]), x0.dtype).at[dst].add(msg)
        acc = acc + current
    final = pl.pallas_call(
        _finish,
        out_shape=jax.ShapeDtypeStruct(acc.shape, acc.dtype),
        grid=(100,),
        in_specs=[pl.BlockSpec((num_nodes // 100, 64), lambda i: (i, jnp.int32(0)))],
        out_specs=pl.BlockSpec((num_nodes // 100, 64), lambda i: (i, jnp.int32(0))),
    )(acc)
    return final[:N_USERS], final[N_USERS:]


# speedup vs baseline: 37.1438x; 6.9152x over previous
import dataclasses
import functools

import jax
import jax.numpy as jnp
from jax import lax
from jax.experimental import pallas as pl
from jax.experimental.pallas import tpu as pltpu
from jax.experimental.pallas import tpu_sc as plsc
from jax._src import config as _jcfg

NU = 50000
NI = 50000
D = 64
NL = 3
E = 800000

NPAD = 51200
NN = 2 * NPAD
DUMMY = NN
ZROWS = NN + 8
EPAD = 819200
EBLK = EPAD // 128
BLK_PER_TILE = EBLK // 16
MACROS = BLK_PER_TILE // 16
T_ROWS = 103424
TZ_PER_TILE = T_ROWS // 16
ZCH = 404
DEG_ROWS = 102912
DEGZ_PER_TILE = DEG_ROWS // 16
HALF_PER_TILE = NPAD // 16
SCALE_PER_TILE = NN // 16
CW = 16

_mesh = plsc.VectorSubcoreMesh(core_axis_name="c", subcore_axis_name="s")
_f32 = jnp.float32

_cp = pltpu.CompilerParams(use_tc_tiling_on_sc=False)
if "needs_layout_passes" in pltpu.CompilerParams.__dataclass_fields__:
    _cp = dataclasses.replace(_cp, needs_layout_passes=False)


def _bcast(ref, r):
    return plsc.load_gather(ref, [jnp.full((CW,), r, jnp.int32)])


def _rsqrt16(x):
    i = plsc.bitcast(x, jnp.int32)
    y = plsc.bitcast(jnp.int32(0x5F3759DF) - (i >> 1), _f32)
    for _ in range(3):
        y = y * (1.5 - 0.5 * x * y * y)
    return y


def _k0_body(row2d, cola2d, xu, xi, dinv2_o, dsq_o, z0, z1, z2, z3,
             deg_sh, zbuf, ibuf, ones, dbuf, ybuf, y2buf, sqbuf, xbuf,
             zb0, zb1, zb2, zb3):
    c = jnp.int32(lax.axis_index("c"))
    s = jnp.int32(lax.axis_index("s"))
    zbs = (zb0, zb1, zb2, zb3)
    zouts = (z0, z1, z2, z3)

    @pl.loop(0, DEGZ_PER_TILE // CW)
    def _(i):
        i = jnp.int32(i)
        zbuf[pl.ds(i * CW, CW)] = jnp.zeros((CW,), _f32)

    @pl.loop(0, 128 // CW)
    def _(i):
        i = jnp.int32(i)
        ones[pl.ds(i * CW, CW)] = jnp.ones((CW,), _f32)

    pltpu.sync_copy(zbuf, deg_sh.at[pl.ds(s * DEGZ_PER_TILE, DEGZ_PER_TILE)])
    plsc.subcore_barrier()

    def deg_pass(idx2d):
        @pl.loop(0, MACROS)
        def _(m):
            m = jnp.int32(m)
            pltpu.sync_copy(idx2d.at[pl.ds((s * MACROS + m) * 16, 16)], ibuf)

            @pl.loop(0, 16)
            def _(j):
                j = jnp.int32(j)
                pltpu.sync_copy(ones, deg_sh.at[ibuf.at[j]], add=True)

    @pl.when(c == 0)
    def _():
        deg_pass(row2d)

    @pl.when(c == 1)
    def _():
        deg_pass(cola2d)

    plsc.subcore_barrier()

    def half_pass(x_table, node_base):
        @pl.loop(0, HALF_PER_TILE // 128)
        def _(b):
            b = jnp.int32(b)
            loc0 = s * HALF_PER_TILE + b * 128
            g0 = node_base + loc0
            pltpu.sync_copy(deg_sh.at[pl.ds(g0, 128)], dbuf)

            @pl.loop(0, 128 // CW)
            def _(i):
                i = jnp.int32(i)
                x = jnp.maximum(dbuf[pl.ds(i * CW, CW)], 1.0)
                y = _rsqrt16(x)
                ybuf[pl.ds(i * CW, CW)] = y
                y2buf[pl.ds(i * CW, CW)] = y * y
                sqbuf[pl.ds(i * CW, CW)] = x * y

            pltpu.sync_copy(y2buf, dinv2_o.at[pl.ds(g0, 128)])
            pltpu.sync_copy(sqbuf, dsq_o.at[pl.ds(g0, 128)])
            pltpu.sync_copy(x_table.at[pl.ds(loc0, 128)], xbuf)

            @pl.loop(0, 128)
            def _(r):
                r = jnp.int32(r)
                dv = _bcast(ybuf, r)
                for k in range(4):
                    zbs[k][r] = xbuf[r, pl.ds(k * CW, CW)] * dv

            for k in range(4):
                pltpu.sync_copy(zbs[k], zouts[k].at[pl.ds(g0, 128)])

    @pl.when(c == 0)
    def _():
        half_pass(xu, 0)

    @pl.when(c == 1)
    def _():
        half_pass(xi, NPAD)


def _layer_body(last, row2d, cola2d, zi0, zi1, zi2, zi3, si0, si1, si2, si3,
                dinv2, dsq, *rest):
    if last:
        outs = rest[:4]
        rest = rest[4:]
    else:
        zo = rest[:4]
        so = rest[4:8]
        rest = rest[8:]
    (t_sh, zrow, ribuf, cibuf, gbuf, tbuf, sbuf, dbuf, qbuf, b1, b2) = rest
    c = jnp.int32(lax.axis_index("c"))
    s = jnp.int32(lax.axis_index("s"))
    zis = (zi0, zi1, zi2, zi3)
    sis = (si0, si1, si2, si3)

    @pl.loop(0, ZCH)
    def _(i):
        i = jnp.int32(i)
        zrow[i] = jnp.zeros((CW,), _f32)

    def edge_pass(zck):
        @pl.loop(0, MACROS)
        def _(m):
            m = jnp.int32(m)
            blk0 = (s * MACROS + m) * 16
            pltpu.sync_copy(row2d.at[pl.ds(blk0, 16)], ribuf)
            pltpu.sync_copy(cola2d.at[pl.ds(blk0, 16)], cibuf)

            @pl.loop(0, 16)
            def _(j):
                j = jnp.int32(j)
                pltpu.sync_copy(zck.at[ribuf.at[j]], gbuf)
                pltpu.sync_copy(gbuf, t_sh.at[cibuf.at[j]], add=True)
                pltpu.sync_copy(zck.at[cibuf.at[j]], gbuf)
                pltpu.sync_copy(gbuf, t_sh.at[ribuf.at[j]], add=True)

    def scale_pass(k):
        @pl.loop(0, SCALE_PER_TILE // 128)
        def _(b):
            b = jnp.int32(b)
            row0 = s * SCALE_PER_TILE + b * 128
            pltpu.sync_copy(t_sh.at[pl.ds(row0, 128)], tbuf)
            pltpu.sync_copy(dinv2.at[pl.ds(row0, 128)], dbuf)
            pltpu.sync_copy(sis[k].at[pl.ds(row0, 128)], sbuf)
            if last:
                pltpu.sync_copy(dsq.at[pl.ds(row0, 128)], qbuf)

            @pl.loop(0, 128)
            def _(r):
                r = jnp.int32(r)
                dv = _bcast(dbuf, r)
                zv = tbuf[r] * dv
                sv = sbuf[r] + zv
                if last:
                    b1[r] = sv * _bcast(qbuf, r) * 0.25
                else:
                    b1[r] = zv
                    b2[r] = sv

            if last:
                pltpu.sync_copy(b1, outs[k].at[pl.ds(row0, 128)])
            else:
                pltpu.sync_copy(b1, zo[k].at[pl.ds(row0, 128)])
                pltpu.sync_copy(b2, so[k].at[pl.ds(row0, 128)])

    for p in range(2):
        @pl.loop(0, TZ_PER_TILE // ZCH)
        def _(i):
            i = jnp.int32(i)
            pltpu.sync_copy(
                zrow, t_sh.at[pl.ds(s * TZ_PER_TILE + i * ZCH, ZCH)])

        plsc.subcore_barrier()

        @pl.when(c == 0)
        def _():
            edge_pass(zis[p])

        @pl.when(c == 1)
        def _():
            edge_pass(zis[2 + p])

        plsc.subcore_barrier()

        @pl.when(c == 0)
        def _():
            scale_pass(p)

        @pl.when(c == 1)
        def _():
            scale_pass(2 + p)

        plsc.subcore_barrier()


def _zc(shape):
    return jax.ShapeDtypeStruct(shape, _f32)


_k0 = pl.kernel(
    _k0_body, mesh=_mesh, compiler_params=_cp,
    out_type=(_zc((NN,)), _zc((NN,))) + tuple(_zc((ZROWS, CW)) for _ in range(4)),
    scratch_types=[
        pltpu.VMEM_SHARED((DEG_ROWS,), _f32),
        pltpu.VMEM((DEGZ_PER_TILE,), _f32),
        pltpu.VMEM((16, 128), jnp.int32),
        pltpu.VMEM((128,), _f32),
        pltpu.VMEM((128,), _f32),
        pltpu.VMEM((128,), _f32),
        pltpu.VMEM((128,), _f32),
        pltpu.VMEM((128,), _f32),
        pltpu.VMEM((128, D), _f32),
    ] + [pltpu.VMEM((128, CW), _f32) for _ in range(4)],
)


def _layer(last):
    if last:
        out_type = tuple(_zc((NN, CW)) for _ in range(4))
    else:
        out_type = (tuple(_zc((ZROWS, CW)) for _ in range(4))
                    + tuple(_zc((NN, CW)) for _ in range(4)))
    return pl.kernel(
        functools.partial(_layer_body, last), mesh=_mesh, compiler_params=_cp,
        out_type=out_type,
        scratch_types=[
            pltpu.VMEM_SHARED((T_ROWS, CW), _f32),
            pltpu.VMEM((ZCH, CW), _f32),
            pltpu.VMEM((16, 128), jnp.int32),
            pltpu.VMEM((16, 128), jnp.int32),
            pltpu.VMEM((128, CW), _f32),
            pltpu.VMEM((128, CW), _f32),
            pltpu.VMEM((128, CW), _f32),
            pltpu.VMEM((128,), _f32),
            pltpu.VMEM((128,), _f32),
            pltpu.VMEM((128, CW), _f32),
            pltpu.VMEM((128, CW), _f32),
        ],
    )


def kernel(edge_index, user_table, item_table):
    with _jcfg.enable_x64(False):
        return _kernel_x32(edge_index, user_table, item_table)


def _kernel_x32(edge_index, user_table, item_table):
    row = edge_index[0].astype(jnp.int32)
    col = edge_index[1].astype(jnp.int32) + NPAD
    pad = jnp.full((EPAD - E,), DUMMY, jnp.int32)
    row2d = jnp.concatenate([row, pad]).reshape(EBLK, 128)
    cola2d = jnp.concatenate([col, pad]).reshape(EBLK, 128)
    xu = jnp.pad(user_table, ((0, NPAD - NU), (0, 0)))
    xi = jnp.pad(item_table, ((0, NPAD - NI), (0, 0)))

    dinv2, dsq, *z0 = _k0(row2d, cola2d, xu, xi)
    step = _layer(False)
    r1 = step(row2d, cola2d, *z0, *z0, dinv2, dsq)
    z1, s1 = r1[:4], r1[4:]
    r2 = step(row2d, cola2d, *z1, *s1, dinv2, dsq)
    z2, s2 = r2[:4], r2[4:]
    outs = _layer(True)(row2d, cola2d, *z2, *s2, dinv2, dsq)

    out = jnp.concatenate(outs, axis=1)
    return out[:NU], out[NPAD:NPAD + NI]


# speedup vs baseline: 66.8691x; 1.8003x over previous
import dataclasses
import functools

import jax
import jax.numpy as jnp
from jax import lax
from jax.experimental import pallas as pl
from jax.experimental.pallas import tpu as pltpu
from jax.experimental.pallas import tpu_sc as plsc
from jax._src import config as _jcfg

NU = 50000
NI = 50000
D = 64
NL = 3
E = 800000

NPAD = 51200
NN = 2 * NPAD
DUMMY = NN
ZROWS = NN + 8
EPAD = 819200
EBLK = EPAD // 128
BLK_PER_TILE = EBLK // 16
MACROS = BLK_PER_TILE // 16
T_ROWS = 102528
TZ_PER_TILE = NN // 16
DEG_ROWS = 102912
DEGZ_PER_TILE = DEG_ROWS // 16
HALF_PER_TILE = NPAD // 16
SCALE_PER_TILE = NN // 16
CW = 16
NSLOT = 8
LEAD = 4
MAC_BLK = 16
MACROS2 = BLK_PER_TILE // MAC_BLK

_mesh = plsc.VectorSubcoreMesh(core_axis_name="c", subcore_axis_name="s")
_f32 = jnp.float32

_cp = pltpu.CompilerParams(use_tc_tiling_on_sc=False)
if "needs_layout_passes" in pltpu.CompilerParams.__dataclass_fields__:
    _cp = dataclasses.replace(_cp, needs_layout_passes=False)


def _bcast(ref, r):
    return plsc.load_gather(ref, [jnp.full((CW,), r, jnp.int32)])


def _rsqrt16(x):
    i = plsc.bitcast(x, jnp.int32)
    y = plsc.bitcast(jnp.int32(0x5F3759DF) - (i >> 1), _f32)
    for _ in range(3):
        y = y * (1.5 - 0.5 * x * y * y)
    return y


def _k0_body(row2d, cola2d, xu, xi, dinv2_o, dsq_o, z0, z1, z2, z3,
             deg_sh, zbuf, ibuf, ones, dbuf, ybuf, y2buf, sqbuf, xbuf,
             zb0, zb1, zb2, zb3):
    c = jnp.int32(lax.axis_index("c"))
    s = jnp.int32(lax.axis_index("s"))
    zbs = (zb0, zb1, zb2, zb3)
    zouts = (z0, z1, z2, z3)

    @pl.loop(0, DEGZ_PER_TILE // CW)
    def _(i):
        i = jnp.int32(i)
        zbuf[pl.ds(i * CW, CW)] = jnp.zeros((CW,), _f32)

    @pl.loop(0, 128 // CW)
    def _(i):
        i = jnp.int32(i)
        ones[pl.ds(i * CW, CW)] = jnp.ones((CW,), _f32)

    pltpu.sync_copy(zbuf, deg_sh.at[pl.ds(s * DEGZ_PER_TILE, DEGZ_PER_TILE)])
    plsc.subcore_barrier()

    def deg_pass(idx2d):
        @pl.loop(0, MACROS)
        def _(m):
            m = jnp.int32(m)
            pltpu.sync_copy(idx2d.at[pl.ds((s * MACROS + m) * 16, 16)], ibuf)

            @pl.loop(0, 16)
            def _(j):
                j = jnp.int32(j)
                pltpu.sync_copy(ones, deg_sh.at[ibuf.at[j]], add=True)

    @pl.when(c == 0)
    def _():
        deg_pass(row2d)

    @pl.when(c == 1)
    def _():
        deg_pass(cola2d)

    plsc.subcore_barrier()

    def half_pass(x_table, node_base):
        @pl.loop(0, HALF_PER_TILE // 128)
        def _(b):
            b = jnp.int32(b)
            loc0 = s * HALF_PER_TILE + b * 128
            g0 = node_base + loc0
            pltpu.sync_copy(deg_sh.at[pl.ds(g0, 128)], dbuf)

            @pl.loop(0, 128 // CW)
            def _(i):
                i = jnp.int32(i)
                x = jnp.maximum(dbuf[pl.ds(i * CW, CW)], 1.0)
                y = _rsqrt16(x)
                ybuf[pl.ds(i * CW, CW)] = y
                y2buf[pl.ds(i * CW, CW)] = y * y
                sqbuf[pl.ds(i * CW, CW)] = x * y

            pltpu.sync_copy(y2buf, dinv2_o.at[pl.ds(g0, 128)])
            pltpu.sync_copy(sqbuf, dsq_o.at[pl.ds(g0, 128)])
            pltpu.sync_copy(x_table.at[pl.ds(loc0, 128)], xbuf)

            @pl.loop(0, 128)
            def _(r):
                r = jnp.int32(r)
                dv = _bcast(ybuf, r)
                for k in range(4):
                    zbs[k][r] = xbuf[r, pl.ds(k * CW, CW)] * dv

            for k in range(4):
                pltpu.sync_copy(zbs[k], zouts[k].at[pl.ds(g0, 128)])

    @pl.when(c == 0)
    def _():
        half_pass(xu, 0)

    @pl.when(c == 1)
    def _():
        half_pass(xi, NPAD)


def _layer_body(last, row2d, cola2d, zi0, zi1, zi2, zi3, si0, si1, si2, si3,
                dinv2, dsq, *rest):
    if last:
        outs = rest[:4]
        rest = rest[4:]
    else:
        zo = rest[:4]
        so = rest[4:8]
        rest = rest[8:]
    t_sh, ribuf, cibuf = rest[:3]
    gbs = rest[3:3 + NSLOT]
    rest = rest[3 + NSLOT:]
    (tbA, sbA, o1A, o2A, tbB, sbB, o1B, o2B) = gbs
    (dbA, qbA, dbB, qbB) = rest[:4]
    sems = rest[4:]
    gsems = sems[:NSLOT]
    ssems = sems[NSLOT:2 * NSLOT]
    zsem, isemA, isemB, osemA, osemB = sems[2 * NSLOT:]
    c = jnp.int32(lax.axis_index("c"))
    s = jnp.int32(lax.axis_index("s"))
    zis = (zi0, zi1, zi2, zi3)
    sis = (si0, si1, si2, si3)

    def zero_t():
        @pl.loop(0, 128)
        def _(r):
            r = jnp.int32(r)
            for gb in gbs:
                gb[r] = jnp.zeros((CW,), _f32)

        descs = {}
        n = TZ_PER_TILE // 128
        for i in range(n + NSLOT):
            if i < n:
                if i >= NSLOT:
                    descs.pop(i - NSLOT).wait()
                descs[i] = pltpu.async_copy(
                    gbs[i % NSLOT],
                    t_sh.at[pl.ds(s * TZ_PER_TILE + i * 128, 128)],
                    ssems[i % NSLOT])
            elif (i - NSLOT) in descs:
                descs.pop(i - NSLOT).wait()

    def edge_pass(zck):
        @pl.loop(0, MACROS2)
        def _(m):
            m = jnp.int32(m)
            blk0 = s * BLK_PER_TILE + m * MAC_BLK
            pltpu.sync_copy(row2d.at[pl.ds(blk0, MAC_BLK)], ribuf)
            pltpu.sync_copy(cola2d.at[pl.ds(blk0, MAC_BLK)], cibuf)

            @pl.loop(0, MAC_BLK, step=4)
            def _(j):
                j = jnp.int32(j)
                dg, dsc = {}, {}
                for u in range(4):
                    for d in range(2):
                        slot = u * 2 + d
                        gi = ribuf if d == 0 else cibuf
                        dg[slot] = pltpu.async_copy(
                            zck.at[gi.at[j + u]], gbs[slot], gsems[slot])
                for u in range(4):
                    for d in range(2):
                        slot = u * 2 + d
                        si_ = cibuf if d == 0 else ribuf
                        dg[slot].wait()
                        dsc[slot] = pltpu.async_copy(
                            gbs[slot], t_sh.at[si_.at[j + u]],
                            ssems[slot], add=True)
                for slot in range(NSLOT):
                    dsc[slot].wait()

    def scale_pass(k):
        def issue_in(row0, tb, sb, db, qb, sems4):
            ds_ = [pltpu.async_copy(t_sh.at[pl.ds(row0, 128)], tb, sems4[0]),
                   pltpu.async_copy(dinv2.at[pl.ds(row0, 128)], db, sems4[1]),
                   pltpu.async_copy(sis[k].at[pl.ds(row0, 128)], sb, sems4[2])]
            if last:
                ds_.append(
                    pltpu.async_copy(dsq.at[pl.ds(row0, 128)], qb, sems4[3]))
            return ds_

        def compute(tb, sb, db, qb, o1, o2):
            @pl.loop(0, 128 // 8)
            def _(i):
                i = jnp.int32(i)
                for u in range(8):
                    r = i * 8 + u
                    dv = _bcast(db, r)
                    zv = tb[r] * dv
                    sv = sb[r] + zv
                    if last:
                        o1[r] = sv * _bcast(qb, r) * 0.25
                    else:
                        o1[r] = zv
                        o2[r] = sv

        def issue_out(row0, o1, o2, sems2):
            if last:
                return [pltpu.async_copy(
                    o1, outs[k].at[pl.ds(row0, 128)], sems2[0])]
            return [pltpu.async_copy(o1, zo[k].at[pl.ds(row0, 128)], sems2[0]),
                    pltpu.async_copy(o2, so[k].at[pl.ds(row0, 128)], sems2[1])]

        @pl.loop(0, SCALE_PER_TILE // 256)
        def _(b):
            b = jnp.int32(b)
            row0 = s * SCALE_PER_TILE + b * 256
            inA = issue_in(row0, tbA, sbA, dbA, qbA, gsems[:4])
            inB = issue_in(row0 + 128, tbB, sbB, dbB, qbB, gsems[4:])
            for d in inA:
                d.wait()
            compute(tbA, sbA, dbA, qbA, o1A, o2A)
            outA = issue_out(row0, o1A, o2A, ssems[:2])
            for d in inB:
                d.wait()
            compute(tbB, sbB, dbB, qbB, o1B, o2B)
            outB = issue_out(row0 + 128, o1B, o2B, ssems[2:4])
            for d in outA + outB:
                d.wait()

    for p in range(2):
        zero_t()
        plsc.subcore_barrier()

        @pl.when(c == 0)
        def _():
            edge_pass(zis[p])

        @pl.when(c == 1)
        def _():
            edge_pass(zis[2 + p])

        plsc.subcore_barrier()

        @pl.when(c == 0)
        def _():
            scale_pass(p)

        @pl.when(c == 1)
        def _():
            scale_pass(2 + p)

        plsc.subcore_barrier()


def _zc(shape):
    return jax.ShapeDtypeStruct(shape, _f32)


_k0 = pl.kernel(
    _k0_body, mesh=_mesh, compiler_params=_cp,
    out_type=(_zc((NN,)), _zc((NN,))) + tuple(_zc((ZROWS, CW)) for _ in range(4)),
    scratch_types=[
        pltpu.VMEM_SHARED((DEG_ROWS,), _f32),
        pltpu.VMEM((DEGZ_PER_TILE,), _f32),
        pltpu.VMEM((16, 128), jnp.int32),
        pltpu.VMEM((128,), _f32),
        pltpu.VMEM((128,), _f32),
        pltpu.VMEM((128,), _f32),
        pltpu.VMEM((128,), _f32),
        pltpu.VMEM((128,), _f32),
        pltpu.VMEM((128, D), _f32),
    ] + [pltpu.VMEM((128, CW), _f32) for _ in range(4)],
)


def _layer(last):
    if last:
        out_type = tuple(_zc((NN, CW)) for _ in range(4))
    else:
        out_type = (tuple(_zc((ZROWS, CW)) for _ in range(4))
                    + tuple(_zc((NN, CW)) for _ in range(4)))
    return pl.kernel(
        functools.partial(_layer_body, last), mesh=_mesh, compiler_params=_cp,
        out_type=out_type,
        scratch_types=(
            [pltpu.VMEM_SHARED((T_ROWS, CW), _f32),
             pltpu.VMEM((MAC_BLK, 128), jnp.int32),
             pltpu.VMEM((MAC_BLK, 128), jnp.int32)]
            + [pltpu.VMEM((128, CW), _f32) for _ in range(NSLOT)]
            + [pltpu.VMEM((128,), _f32)] * 4
            + [pltpu.SemaphoreType.DMA] * (2 * NSLOT + 5)
        ),
    )


def kernel(edge_index, user_table, item_table):
    with _jcfg.enable_x64(False):
        return _kernel_x32(edge_index, user_table, item_table)


def _kernel_x32(edge_index, user_table, item_table):
    row = edge_index[0].astype(jnp.int32)
    col = edge_index[1].astype(jnp.int32) + NPAD
    pad = jnp.full((EPAD - E,), DUMMY, jnp.int32)
    row2d = jnp.concatenate([row, pad]).reshape(EBLK, 128)
    cola2d = jnp.concatenate([col, pad]).reshape(EBLK, 128)
    xu = jnp.pad(user_table, ((0, NPAD - NU), (0, 0)))
    xi = jnp.pad(item_table, ((0, NPAD - NI), (0, 0)))

    dinv2, dsq, *z0 = _k0(row2d, cola2d, xu, xi)
    step = _layer(False)
    r1 = step(row2d, cola2d, *z0, *z0, dinv2, dsq)
    z1, s1 = r1[:4], r1[4:]
    r2 = step(row2d, cola2d, *z1, *s1, dinv2, dsq)
    z2, s2 = r2[:4], r2[4:]
    outs = _layer(True)(row2d, cola2d, *z2, *s2, dinv2, dsq)

    out = jnp.concatenate(outs, axis=1)
    return out[:NU], out[NPAD:NPAD + NI]
